# baseline (speedup 1.0000x reference)
import functools

import jax
import jax.numpy as jnp
from jax import lax
from jax.experimental import pallas as pl
from jax.experimental.pallas import tpu as pltpu
from jax.experimental.pallas import tpu_sc as plsc

_E = 8
_K = 2
_T = 2048
_H = 1024
_I = 1024
_BLK = 128
_NB = 39
_S = _NB * _BLK

_NC = 2
_NS = 16
_NW = _NC * _NS
_TPW = _T // _NW
_CH4 = 16


def _cumsum0(x):
    n = x.shape[0]
    sh = 1
    while sh < n:
        x = x + jnp.concatenate(
            [jnp.zeros((sh,) + x.shape[1:], x.dtype), x[:-sh]], axis=0)
        sh *= 2
    return x


def _cumsum_tokens(oh):
    nc = _T // _BLK
    ohc = oh.reshape(nc, _BLK, _E).astype(jnp.float32)
    row = lax.broadcasted_iota(jnp.int32, (_BLK, _BLK), 0)
    col = lax.broadcasted_iota(jnp.int32, (_BLK, _BLK), 1)
    tri = (row >= col).astype(jnp.float32)
    trib = jnp.broadcast_to(tri[None], (nc, _BLK, _BLK))
    incl = lax.dot_general(trib, ohc, (((2,), (1,)), ((0,), (0,))),
                           preferred_element_type=jnp.float32)
    tot = incl[:, _BLK - 1, :]
    cpre = _cumsum0(tot) - tot
    return (incl + cpre[:, None, :]).reshape(_T, _E).astype(jnp.int32)


def _routing_body(logits_ref, dest0_ref, dest1_ref, w0_ref, w1_ref, meta_ref):
    logits = logits_ref[...]
    m = jnp.max(logits, axis=1, keepdims=True)
    ex = jnp.exp(logits - m)
    probs = ex / jnp.sum(ex, axis=1, keepdims=True)
    lane = lax.broadcasted_iota(jnp.int32, (_T, _E), 1)
    m1 = jnp.max(probs, axis=1, keepdims=True)
    i1 = jnp.min(jnp.where(probs == m1, lane, _E), axis=1, keepdims=True)
    masked = jnp.where(lane == i1, -jnp.inf, probs)
    m2 = jnp.max(masked, axis=1, keepdims=True)
    i2 = jnp.min(jnp.where(masked == m2, lane, _E), axis=1, keepdims=True)
    ssum = m1 + m2
    w0_ref[...] = jnp.broadcast_to(m1 / ssum, (_T, 16))
    w1_ref[...] = jnp.broadcast_to(m2 / ssum, (_T, 16))

    oh1 = (lane == i1).astype(jnp.int32)
    oh2 = (lane == i2).astype(jnp.int32)
    c1 = jnp.sum(oh1, axis=0, keepdims=True)
    cnt = c1 + jnp.sum(oh2, axis=0, keepdims=True)
    rank1 = _cumsum_tokens(oh1) - oh1
    rank2 = c1 + _cumsum_tokens(oh2) - oh2
    pcnt = ((cnt + (_BLK - 1)) // _BLK) * _BLK
    inc = pcnt
    sh = 1
    while sh < _E:
        inc = inc + jnp.concatenate(
            [jnp.zeros((1, sh), jnp.int32), inc[:, :-sh]], axis=1)
        sh *= 2
    poff = inc - pcnt
    dest0_ref[...] = jnp.sum(oh1 * (poff + rank1), axis=1)
    dest1_ref[...] = jnp.sum(oh2 * (poff + rank2), axis=1)
    bid = lax.broadcasted_iota(jnp.int32, (_NB, _E), 0)
    start = poff // _BLK
    bev = jnp.sum((bid >= start).astype(jnp.int32), axis=1) - 1
    isst = jnp.concatenate(
        [jnp.ones((1,), jnp.int32), (bev[1:] != bev[:-1]).astype(jnp.int32)])
    ridx = _cumsum0(isst) - 1
    slot = ridx & 1
    nxtmask = ((ridx[None, :] == ridx[:, None] + 1)
               & (isst[None, :] == 1)).astype(jnp.int32)
    nxt = jnp.sum(nxtmask * bev[None, :], axis=1)
    hasnxt = jnp.sum(nxtmask, axis=1)
    meta_ref[...] = jnp.concatenate(
        [bev[None], isst[None], slot[None], nxt[None], hasnxt[None],
         jnp.zeros((3, _NB), jnp.int32)], axis=0)


def _gemm_body(meta_ref, x_ref, w13_hbm, w2_hbm, out_ref,
               w13_buf, w2_buf, sem13, sem2):
    b = pl.program_id(0)
    slot = meta_ref[2, b]

    def copies(e, s):
        cps = []
        for j in range(4):
            cps.append(pltpu.make_async_copy(
                w13_hbm.at[e, pl.ds(j * (_I // 2), _I // 2)],
                w13_buf.at[s, pl.ds(j * (_I // 2), _I // 2)],
                sem13.at[s, j]))
        for j in range(2):
            cps.append(pltpu.make_async_copy(
                w2_hbm.at[e, pl.ds(j * (_H // 2), _H // 2)],
                w2_buf.at[s, pl.ds(j * (_H // 2), _H // 2)],
                sem2.at[s, j]))
        return cps

    @pl.when(b == 0)
    def _():
        for cp in copies(meta_ref[0, 0], 0):
            cp.start()

    @pl.when(meta_ref[1, b] == 1)
    def _():
        for cp in copies(meta_ref[0, b], slot):
            cp.wait()

    @pl.when((meta_ref[1, b] == 1) & (meta_ref[4, b] == 1))
    def _():
        for cp in copies(meta_ref[3, b], 1 - slot):
            cp.start()

    x = x_ref[...].astype(jnp.bfloat16)
    gu = lax.dot_general(x, w13_buf[slot], (((1,), (1,)), ((), ())),
                         preferred_element_type=jnp.float32)
    gate = gu[:, :_I]
    up = gu[:, _I:]
    act = (gate * jax.nn.sigmoid(gate) * up).astype(jnp.bfloat16)
    down = lax.dot_general(act, w2_buf[slot], (((1,), (1,)), ((), ())),
                           preferred_element_type=jnp.float32)

    def rne16(x):
        u = lax.bitcast_convert_type(x, jnp.int32)
        return lax.shift_right_logical(
            u + 0x7FFF + (lax.shift_right_logical(u, 16) & 1), 16)

    out_ref[...] = rne16(down[:, :_H // 2]) | (rne16(down[:, _H // 2:]) << 16)


def _scatter_body(hidden_hbm, dest0_hbm, dest1_hbm, xs_hbm,
                  d0_v, d1_v, rows_v, sem0, sem1):
    wid = lax.axis_index("s") * _NC + lax.axis_index("c")
    base = wid * _TPW
    pltpu.sync_copy(dest0_hbm.at[pl.ds(base, _TPW)], d0_v)
    pltpu.sync_copy(dest1_hbm.at[pl.ds(base, _TPW)], d1_v)
    pltpu.sync_copy(hidden_hbm.at[pl.ds(base, _TPW)], rows_v)
    cp0 = pltpu.async_copy(rows_v, xs_hbm.at[d0_v], sem0)
    cp1 = pltpu.async_copy(rows_v, xs_hbm.at[d1_v], sem1)
    cp0.wait()
    cp1.wait()


def _combine_body(down_hbm, dest0_hbm, dest1_hbm, w0_hbm, w1_hbm, out_hbm,
                  d0_v, d1_v, w_v, r0_v, r1_v, ob_v, sems, wsems):
    wid = lax.axis_index("s") * _NC + lax.axis_index("c")
    base = wid * _TPW
    pltpu.sync_copy(dest0_hbm.at[pl.ds(base, _TPW)], d0_v)
    pltpu.sync_copy(dest1_hbm.at[pl.ds(base, _TPW)], d1_v)
    pltpu.sync_copy(w0_hbm.at[pl.ds(base, _TPW)], w_v.at[0])
    pltpu.sync_copy(w1_hbm.at[pl.ds(base, _TPW)], w_v.at[1])
    nch = _TPW // _CH4

    def gathers(c):
        buf = c % 2
        i0 = d0_v[pl.ds(c * _CH4, _CH4)]
        i1 = d1_v[pl.ds(c * _CH4, _CH4)]
        cp0 = pltpu.async_copy(down_hbm.at[i0], r0_v.at[buf], sems.at[buf, 0])
        cp1 = pltpu.async_copy(down_hbm.at[i1], r1_v.at[buf], sems.at[buf, 1])
        return cp0, cp1

    inflight = gathers(0)
    pend = [None, None]
    for c in range(nch):
        buf = c % 2
        nxt = None
        if c + 1 < nch:
            if pend[1 - buf] is not None:
                pend[1 - buf].wait()
                pend[1 - buf] = None
            nxt = gathers(c + 1)
        inflight[0].wait()
        inflight[1].wait()
        inflight = nxt

        def body(i, carry):
            s0 = w_v[0, c * _CH4 + i, :]
            s1 = w_v[1, c * _CH4 + i, :]
            mask = jnp.int32(-65536)
            for q in range(_H // 32):
                sl = pl.ds(q * 16, 16)
                v0 = r0_v[buf, i, sl]
                v1 = r1_v[buf, i, sl]
                a0 = lax.bitcast_convert_type(v0 << 16, jnp.float32)
                a1 = lax.bitcast_convert_type(v1 << 16, jnp.float32)
                b0 = lax.bitcast_convert_type(v0 & mask, jnp.float32)
                b1 = lax.bitcast_convert_type(v1 & mask, jnp.float32)
                ob_v[buf, i, sl] = s0 * a0 + s1 * a1
                ob_v[buf, i, pl.ds(_H // 2 + q * 16, 16)] = s0 * b0 + s1 * b1
            return carry

        lax.fori_loop(0, _CH4, body, 0)
        pend[buf] = pltpu.async_copy(
            ob_v.at[buf], out_hbm.at[pl.ds(base + c * _CH4, _CH4)],
            wsems.at[buf])
    for p in pend:
        if p is not None:
            p.wait()


def kernel(hidden_states, router_logits, w13_weight, w2_weight):
    dest0, dest1, w0, w1, meta = pl.pallas_call(
        _routing_body,
        out_shape=(
            jax.ShapeDtypeStruct((_T,), jnp.int32),
            jax.ShapeDtypeStruct((_T,), jnp.int32),
            jax.ShapeDtypeStruct((_T, 16), jnp.float32),
            jax.ShapeDtypeStruct((_T, 16), jnp.float32),
            jax.ShapeDtypeStruct((8, _NB), jnp.int32),
        ),
    )(router_logits)

    mesh = plsc.VectorSubcoreMesh(core_axis_name="c", subcore_axis_name="s")

    scatter_k = functools.partial(
        pl.kernel,
        mesh=mesh,
        out_type=jax.ShapeDtypeStruct((_S, _H), jnp.float32),
        scratch_types=[
            pltpu.VMEM((_TPW,), jnp.int32),
            pltpu.VMEM((_TPW,), jnp.int32),
            pltpu.VMEM((_TPW, _H), jnp.float32),
            pltpu.SemaphoreType.DMA,
            pltpu.SemaphoreType.DMA,
        ],
    )(_scatter_body)
    x_sorted = scatter_k(hidden_states, dest0, dest1)

    grid_spec = pltpu.PrefetchScalarGridSpec(
        num_scalar_prefetch=1,
        grid=(_NB,),
        in_specs=[
            pl.BlockSpec((_BLK, _H), lambda b, m: (b, 0)),
            pl.BlockSpec(memory_space=pl.ANY),
            pl.BlockSpec(memory_space=pl.ANY),
        ],
        out_specs=pl.BlockSpec((_BLK, _H // 2), lambda b, m: (b, 0)),
        scratch_shapes=[
            pltpu.VMEM((2, 2 * _I, _H), jnp.float32),
            pltpu.VMEM((2, _H, _I), jnp.float32),
            pltpu.SemaphoreType.DMA((2, 4)),
            pltpu.SemaphoreType.DMA((2, 2)),
        ],
    )
    down_sorted = pl.pallas_call(
        _gemm_body,
        grid_spec=grid_spec,
        out_shape=jax.ShapeDtypeStruct((_S, _H // 2), jnp.int32),
    )(meta, x_sorted, w13_weight, w2_weight)

    combine_k = functools.partial(
        pl.kernel,
        mesh=mesh,
        out_type=jax.ShapeDtypeStruct((_T, _H), jnp.float32),
        scratch_types=[
            pltpu.VMEM((_TPW,), jnp.int32),
            pltpu.VMEM((_TPW,), jnp.int32),
            pltpu.VMEM((2, _TPW, 16), jnp.float32),
            pltpu.VMEM((2, _CH4, _H // 2), jnp.int32),
            pltpu.VMEM((2, _CH4, _H // 2), jnp.int32),
            pltpu.VMEM((2, _CH4, _H), jnp.float32),
            pltpu.SemaphoreType.DMA((2, 2)),
            pltpu.SemaphoreType.DMA((2,)),
        ],
    )(_combine_body)
    return combine_k(down_sorted, dest0, dest1, w0, w1)


# speedup vs baseline: 1.0131x; 1.0131x over previous
import functools

import jax
import jax.numpy as jnp
from jax import lax
from jax.experimental import pallas as pl
from jax.experimental.pallas import tpu as pltpu
from jax.experimental.pallas import tpu_sc as plsc

_E = 8
_K = 2
_T = 2048
_H = 1024
_I = 1024
_BLK = 128
_NB = 39
_S = _NB * _BLK

_NC = 2
_NS = 16
_NW = _NC * _NS
_TPW = _T // _NW
_CH4 = 16


def _cumsum0(x):
    n = x.shape[0]
    sh = 1
    while sh < n:
        x = x + jnp.concatenate(
            [jnp.zeros((sh,) + x.shape[1:], x.dtype), x[:-sh]], axis=0)
        sh *= 2
    return x


def _cumsum_tokens(oh):
    nc = _T // _BLK
    ohc = oh.reshape(nc, _BLK, _E).astype(jnp.float32)
    row = lax.broadcasted_iota(jnp.int32, (_BLK, _BLK), 0)
    col = lax.broadcasted_iota(jnp.int32, (_BLK, _BLK), 1)
    tri = (row >= col).astype(jnp.float32)
    trib = jnp.broadcast_to(tri[None], (nc, _BLK, _BLK))
    incl = lax.dot_general(trib, ohc, (((2,), (1,)), ((0,), (0,))),
                           preferred_element_type=jnp.float32)
    tot = incl[:, _BLK - 1, :]
    cpre = _cumsum0(tot) - tot
    return (incl + cpre[:, None, :]).reshape(_T, _E).astype(jnp.int32)


def _routing_body(logits_ref, dest0_ref, dest1_ref, w0_ref, w1_ref, meta_ref):
    logits = logits_ref[...]
    m = jnp.max(logits, axis=1, keepdims=True)
    ex = jnp.exp(logits - m)
    probs = ex / jnp.sum(ex, axis=1, keepdims=True)
    lane = lax.broadcasted_iota(jnp.int32, (_T, _E), 1)
    m1 = jnp.max(probs, axis=1, keepdims=True)
    i1 = jnp.min(jnp.where(probs == m1, lane, _E), axis=1, keepdims=True)
    masked = jnp.where(lane == i1, -jnp.inf, probs)
    m2 = jnp.max(masked, axis=1, keepdims=True)
    i2 = jnp.min(jnp.where(masked == m2, lane, _E), axis=1, keepdims=True)
    ssum = m1 + m2
    w0_ref[...] = jnp.broadcast_to(m1 / ssum, (_T, 16))
    w1_ref[...] = jnp.broadcast_to(m2 / ssum, (_T, 16))

    oh1 = (lane == i1).astype(jnp.int32)
    oh2 = (lane == i2).astype(jnp.int32)
    c1 = jnp.sum(oh1, axis=0, keepdims=True)
    cnt = c1 + jnp.sum(oh2, axis=0, keepdims=True)
    rank1 = _cumsum_tokens(oh1) - oh1
    rank2 = c1 + _cumsum_tokens(oh2) - oh2
    pcnt = ((cnt + (_BLK - 1)) // _BLK) * _BLK
    inc = pcnt
    sh = 1
    while sh < _E:
        inc = inc + jnp.concatenate(
            [jnp.zeros((1, sh), jnp.int32), inc[:, :-sh]], axis=1)
        sh *= 2
    poff = inc - pcnt
    dest0_ref[...] = jnp.sum(oh1 * (poff + rank1), axis=1)
    dest1_ref[...] = jnp.sum(oh2 * (poff + rank2), axis=1)
    bid = lax.broadcasted_iota(jnp.int32, (_NB, _E), 0)
    start = poff // _BLK
    bev = jnp.sum((bid >= start).astype(jnp.int32), axis=1) - 1
    isst = jnp.concatenate(
        [jnp.ones((1,), jnp.int32), (bev[1:] != bev[:-1]).astype(jnp.int32)])
    ridx = _cumsum0(isst) - 1
    slot = ridx & 1
    nxtmask = ((ridx[None, :] == ridx[:, None] + 1)
               & (isst[None, :] == 1)).astype(jnp.int32)
    nxt = jnp.sum(nxtmask * bev[None, :], axis=1)
    hasnxt = jnp.sum(nxtmask, axis=1)
    meta_ref[...] = jnp.concatenate(
        [bev[None], isst[None], slot[None], nxt[None], hasnxt[None],
         jnp.zeros((3, _NB), jnp.int32)], axis=0)


def _gemm_body(meta_ref, x_ref, w13_hbm, w2_hbm, out_ref,
               w13_buf, w2_buf, sem13, sem2):
    b = pl.program_id(0)
    slot = meta_ref[2, b]

    def copies(e, s):
        return (pltpu.make_async_copy(w13_hbm.at[e], w13_buf.at[s],
                                      sem13.at[s]),
                pltpu.make_async_copy(w2_hbm.at[e], w2_buf.at[s],
                                      sem2.at[s]))

    @pl.when(b == 0)
    def _():
        for cp in copies(meta_ref[0, 0], 0):
            cp.start()

    @pl.when(meta_ref[1, b] == 1)
    def _():
        for cp in copies(meta_ref[0, b], slot):
            cp.wait()

    @pl.when((meta_ref[1, b] == 1) & (meta_ref[4, b] == 1))
    def _():
        for cp in copies(meta_ref[3, b], 1 - slot):
            cp.start()

    x = x_ref[...].astype(jnp.bfloat16)
    gu = lax.dot_general(x, w13_buf[slot], (((1,), (1,)), ((), ())),
                         preferred_element_type=jnp.float32)
    gate = gu[:, :_I]
    up = gu[:, _I:]
    act = (gate * jax.nn.sigmoid(gate) * up).astype(jnp.bfloat16)
    out_ref[...] = lax.dot_general(act, w2_buf[slot], (((1,), (1,)), ((), ())),
                                   preferred_element_type=jnp.float32)


def _scatter_body(hidden_hbm, dest0_hbm, dest1_hbm, xs_hbm,
                  d0_v, d1_v, rows_v, sem0, sem1):
    wid = lax.axis_index("s") * _NC + lax.axis_index("c")
    base = wid * _TPW
    pltpu.sync_copy(dest0_hbm.at[pl.ds(base, _TPW)], d0_v)
    pltpu.sync_copy(dest1_hbm.at[pl.ds(base, _TPW)], d1_v)
    pltpu.sync_copy(hidden_hbm.at[pl.ds(base, _TPW)], rows_v)
    cp0 = pltpu.async_copy(rows_v, xs_hbm.at[d0_v], sem0)
    cp1 = pltpu.async_copy(rows_v, xs_hbm.at[d1_v], sem1)
    cp0.wait()
    cp1.wait()


def _combine_body(down_hbm, dest0_hbm, dest1_hbm, w0_hbm, w1_hbm, out_hbm,
                  d0_v, d1_v, w_v, r0_v, r1_v, ob_v, sems, wsems):
    wid = lax.axis_index("s") * _NC + lax.axis_index("c")
    base = wid * _TPW
    pltpu.sync_copy(dest0_hbm.at[pl.ds(base, _TPW)], d0_v)
    pltpu.sync_copy(dest1_hbm.at[pl.ds(base, _TPW)], d1_v)
    pltpu.sync_copy(w0_hbm.at[pl.ds(base, _TPW)], w_v.at[0])
    pltpu.sync_copy(w1_hbm.at[pl.ds(base, _TPW)], w_v.at[1])
    nch = _TPW // _CH4

    def gathers(c):
        buf = c % 2
        i0 = d0_v[pl.ds(c * _CH4, _CH4)]
        i1 = d1_v[pl.ds(c * _CH4, _CH4)]
        cp0 = pltpu.async_copy(down_hbm.at[i0], r0_v.at[buf], sems.at[buf, 0])
        cp1 = pltpu.async_copy(down_hbm.at[i1], r1_v.at[buf], sems.at[buf, 1])
        return cp0, cp1

    inflight = gathers(0)
    pend = [None, None]
    for c in range(nch):
        buf = c % 2
        nxt = None
        if c + 1 < nch:
            if pend[1 - buf] is not None:
                pend[1 - buf].wait()
                pend[1 - buf] = None
            nxt = gathers(c + 1)
        inflight[0].wait()
        inflight[1].wait()
        inflight = nxt

        def body(i, carry):
            s0 = w_v[0, c * _CH4 + i, :]
            s1 = w_v[1, c * _CH4 + i, :]
            for q in range(_H // 16):
                sl = pl.ds(q * 16, 16)
                ob_v[buf, i, sl] = s0 * r0_v[buf, i, sl] + s1 * r1_v[buf, i, sl]
            return carry

        lax.fori_loop(0, _CH4, body, 0)
        pend[buf] = pltpu.async_copy(
            ob_v.at[buf], out_hbm.at[pl.ds(base + c * _CH4, _CH4)],
            wsems.at[buf])
    for p in pend:
        if p is not None:
            p.wait()


def kernel(hidden_states, router_logits, w13_weight, w2_weight):
    dest0, dest1, w0, w1, meta = pl.pallas_call(
        _routing_body,
        out_shape=(
            jax.ShapeDtypeStruct((_T,), jnp.int32),
            jax.ShapeDtypeStruct((_T,), jnp.int32),
            jax.ShapeDtypeStruct((_T, 16), jnp.float32),
            jax.ShapeDtypeStruct((_T, 16), jnp.float32),
            jax.ShapeDtypeStruct((8, _NB), jnp.int32),
        ),
    )(router_logits)

    mesh = plsc.VectorSubcoreMesh(core_axis_name="c", subcore_axis_name="s")

    scatter_k = functools.partial(
        pl.kernel,
        mesh=mesh,
        out_type=jax.ShapeDtypeStruct((_S, _H), jnp.float32),
        scratch_types=[
            pltpu.VMEM((_TPW,), jnp.int32),
            pltpu.VMEM((_TPW,), jnp.int32),
            pltpu.VMEM((_TPW, _H), jnp.float32),
            pltpu.SemaphoreType.DMA,
            pltpu.SemaphoreType.DMA,
        ],
    )(_scatter_body)
    x_sorted = scatter_k(hidden_states, dest0, dest1)

    grid_spec = pltpu.PrefetchScalarGridSpec(
        num_scalar_prefetch=1,
        grid=(_NB,),
        in_specs=[
            pl.BlockSpec((_BLK, _H), lambda b, m: (b, 0)),
            pl.BlockSpec(memory_space=pl.ANY),
            pl.BlockSpec(memory_space=pl.ANY),
        ],
        out_specs=pl.BlockSpec((_BLK, _H), lambda b, m: (b, 0)),
        scratch_shapes=[
            pltpu.VMEM((2, 2 * _I, _H), jnp.float32),
            pltpu.VMEM((2, _H, _I), jnp.float32),
            pltpu.SemaphoreType.DMA((2,)),
            pltpu.SemaphoreType.DMA((2,)),
        ],
    )
    down_sorted = pl.pallas_call(
        _gemm_body,
        grid_spec=grid_spec,
        out_shape=jax.ShapeDtypeStruct((_S, _H), jnp.float32),
    )(meta, x_sorted, w13_weight, w2_weight)

    combine_k = functools.partial(
        pl.kernel,
        mesh=mesh,
        out_type=jax.ShapeDtypeStruct((_T, _H), jnp.float32),
        scratch_types=[
            pltpu.VMEM((_TPW,), jnp.int32),
            pltpu.VMEM((_TPW,), jnp.int32),
            pltpu.VMEM((2, _TPW, 16), jnp.float32),
            pltpu.VMEM((2, _CH4, _H), jnp.float32),
            pltpu.VMEM((2, _CH4, _H), jnp.float32),
            pltpu.VMEM((2, _CH4, _H), jnp.float32),
            pltpu.SemaphoreType.DMA((2, 2)),
            pltpu.SemaphoreType.DMA((2,)),
        ],
    )(_combine_body)
    return combine_k(down_sorted, dest0, dest1, w0, w1)
